# baseline (speedup 1.0000x reference)
import functools

import jax
import jax.numpy as jnp
from jax import lax
from jax.experimental import pallas as pl
from jax.experimental.pallas import tpu as pltpu
from jax.experimental.pallas import tpu_sc as plsc

B = 16384
K = 8192
D = 256
COMMIT = 0.25

BM = 512
NB = B // BM

NC = 2
NS = 16
NW = NC * NS
ROWS_PER_W = B // NW
CHUNK = 256


GC = 128
NCH = K // GC
RG = 64
NG = BM // RG


def _dist_body(z_ref, cb_ref, idx_ref, loss_ref, acc_ref, esq_ref,
               ze2_ref, zsq_ref):
    i = pl.program_id(0)
    z = z_ref[...]

    @pl.when(i == 0)
    def _():
        cb = cb_ref[...]
        esq_ref[0, :] = jnp.sum(cb ** 2, axis=-1)
        acc_ref[0] = 0.0

    z_sq = jnp.sum(z ** 2, axis=-1, keepdims=True)
    zsq_ref[...] = jnp.broadcast_to(z_sq, (BM, GC))
    ze2_ref[...] = lax.dot_general(z + z, cb_ref[...], (((1,), (1,)), ((), ())),
                                   preferred_element_type=jnp.float32)

    lane = lax.broadcasted_iota(jnp.int32, (1, GC), 1).astype(jnp.float32)

    def row_group(g, _):
        r0 = g * RG
        zsq_g = zsq_ref[pl.ds(r0, RG), :]
        acc_val = jnp.full((RG, GC), jnp.inf, jnp.float32)
        acc_idx = jnp.zeros((RG, GC), jnp.float32)
        for j in range(NCH):
            ze2_j = ze2_ref[pl.ds(r0, RG), j * GC:(j + 1) * GC]
            esq_j = esq_ref[:, j * GC:(j + 1) * GC]
            t = zsq_g + esq_j
            d = t - ze2_j
            m = d < acc_val
            acc_val = jnp.where(m, d, acc_val)
            acc_idx = jnp.where(m, jnp.float32(j), acc_idx)
        mind_g = jnp.min(acc_val, axis=1)
        k_lane = acc_idx * float(GC) + lane
        hit = jnp.where(acc_val == mind_g[:, None], k_lane, float(K))
        idx_g = jnp.min(hit, axis=1).astype(jnp.int32)
        idx_ref[0, pl.ds(r0, RG), :] = idx_g.reshape(RG, 1)
        acc_ref[0] += jnp.sum(mind_g)
        return 0

    lax.fori_loop(0, NG, row_group, 0)

    @pl.when(i == NB - 1)
    def _():
        loss_ref[0, 0] = acc_ref[0] * ((1.0 + COMMIT) / (B * D))


_dist_call = pl.pallas_call(
    _dist_body,
    grid=(NB,),
    in_specs=[
        pl.BlockSpec((BM, D), lambda i: (i, 0)),
        pl.BlockSpec((K, D), lambda i: (0, 0)),
    ],
    out_specs=[
        pl.BlockSpec((1, BM, 1), lambda i: (i, 0, 0)),
        pl.BlockSpec(memory_space=pltpu.SMEM, block_shape=(1, 1),
                     index_map=lambda i: (0, 0)),
    ],
    out_shape=[
        jax.ShapeDtypeStruct((NB, BM, 1), jnp.int32),
        jax.ShapeDtypeStruct((1, 1), jnp.float32),
    ],
    scratch_shapes=[pltpu.SMEM((1,), jnp.float32),
                    pltpu.VMEM((1, K), jnp.float32),
                    pltpu.VMEM((BM, K), jnp.float32),
                    pltpu.VMEM((BM, GC), jnp.float32)],
)


def _gather_body(cb_hbm, idx_hbm, out_hbm, idx_v, rows_v, sem):
    wid = lax.axis_index("s") * NC + lax.axis_index("c")
    base = wid * ROWS_PER_W
    for c in range(ROWS_PER_W // CHUNK):
        off = base + c * CHUNK
        pltpu.sync_copy(idx_hbm.at[pl.ds(off, CHUNK)], idx_v)
        pltpu.async_copy(cb_hbm.at[idx_v], rows_v, sem).wait()
        pltpu.sync_copy(rows_v, out_hbm.at[pl.ds(off, CHUNK)])


@functools.cache
def _gather_call():
    return functools.partial(
        pl.kernel,
        out_type=jax.ShapeDtypeStruct((B, D), jnp.float32),
        mesh=plsc.VectorSubcoreMesh(core_axis_name="c", subcore_axis_name="s"),
        scratch_types=[
            pltpu.VMEM((CHUNK,), jnp.int32),
            pltpu.VMEM((CHUNK, D), jnp.float32),
            pltpu.SemaphoreType.DMA,
        ],
    )(_gather_body)


def kernel(z, codebook):
    idx_blocks, loss = _dist_call(z, codebook)
    indices = idx_blocks.reshape(B)
    z_q_out = _gather_call()(codebook, indices)
    return (z_q_out, loss.reshape(()), indices)


# speedup vs baseline: 1.3032x; 1.3032x over previous
import functools

import jax
import jax.numpy as jnp
from jax import lax
from jax.experimental import pallas as pl
from jax.experimental.pallas import tpu as pltpu
from jax.experimental.pallas import tpu_sc as plsc

B = 16384
K = 8192
D = 256
COMMIT = 0.25

BM = 1024
NB = B // BM

NC = 2
NS = 16
NW = NC * NS
ROWS_PER_W = B // NW
CHUNK = 256


def _dist_body(z_ref, cb_ref, idx_ref, loss_ref, acc_ref, esq_ref):
    i = pl.program_id(0)
    z = z_ref[...]

    @pl.when(i == 0)
    def _():
        cb = cb_ref[...]
        esq_ref[0, :] = jnp.sum(cb ** 2, axis=-1)
        acc_ref[0] = 0.0

    z_sq = jnp.sum(z ** 2, axis=-1, keepdims=True)
    e_sq = esq_ref[...]
    cols = lax.broadcasted_iota(jnp.int32, (1, K), 1).astype(jnp.float32)
    z2 = z + z
    KH = K // 2
    dist = []
    for h in range(2):
        ze2_h = lax.dot_general(z2, cb_ref[h * KH:(h + 1) * KH, :],
                                (((1,), (1,)), ((), ())),
                                preferred_element_type=jnp.float32)
        dist.append(z_sq + e_sq[:, h * KH:(h + 1) * KH] - ze2_h)
    mind_h = [jnp.min(dh, axis=-1) for dh in dist]
    mind = jnp.minimum(mind_h[0], mind_h[1])
    hit = [jnp.where(dist[h] == mind[:, None],
                     cols[:, h * KH:(h + 1) * KH], float(K))
           for h in range(2)]
    idx0 = jnp.min(hit[0], axis=-1)
    idx1 = jnp.min(hit[1], axis=-1)
    idx_ref[0, 0, :] = jnp.minimum(idx0, idx1).astype(jnp.int32)

    acc_ref[0] += jnp.sum(mind)

    @pl.when(i == NB - 1)
    def _():
        loss_ref[0, 0] = acc_ref[0] * ((1.0 + COMMIT) / (B * D))


_dist_call = pl.pallas_call(
    _dist_body,
    grid=(NB,),
    in_specs=[
        pl.BlockSpec((BM, D), lambda i: (i, 0)),
        pl.BlockSpec((K, D), lambda i: (0, 0)),
    ],
    out_specs=[
        pl.BlockSpec((1, 1, BM), lambda i: (i, 0, 0)),
        pl.BlockSpec(memory_space=pltpu.SMEM, block_shape=(1, 1),
                     index_map=lambda i: (0, 0)),
    ],
    out_shape=[
        jax.ShapeDtypeStruct((NB, 1, BM), jnp.int32),
        jax.ShapeDtypeStruct((1, 1), jnp.float32),
    ],
    scratch_shapes=[pltpu.SMEM((1,), jnp.float32),
                    pltpu.VMEM((1, K), jnp.float32)],
)


def _gather_body(cb_hbm, idx_hbm, out_hbm, idx_v, rows_v, sem):
    wid = lax.axis_index("s") * NC + lax.axis_index("c")
    base = wid * ROWS_PER_W
    for c in range(ROWS_PER_W // CHUNK):
        off = base + c * CHUNK
        pltpu.sync_copy(idx_hbm.at[pl.ds(off, CHUNK)], idx_v)
        pltpu.async_copy(cb_hbm.at[idx_v], rows_v, sem).wait()
        pltpu.sync_copy(rows_v, out_hbm.at[pl.ds(off, CHUNK)])


@functools.cache
def _gather_call():
    return functools.partial(
        pl.kernel,
        out_type=jax.ShapeDtypeStruct((B, D), jnp.float32),
        mesh=plsc.VectorSubcoreMesh(core_axis_name="c", subcore_axis_name="s"),
        scratch_types=[
            pltpu.VMEM((CHUNK,), jnp.int32),
            pltpu.VMEM((CHUNK, D), jnp.float32),
            pltpu.SemaphoreType.DMA,
        ],
    )(_gather_body)


def kernel(z, codebook):
    idx_blocks, loss = _dist_call(z, codebook)
    indices = idx_blocks.reshape(B)
    z_q_out = _gather_call()(codebook, indices)
    return (z_q_out, loss.reshape(()), indices)


# speedup vs baseline: 1.3169x; 1.0105x over previous
import functools

import jax
import jax.numpy as jnp
from jax import lax
from jax.experimental import pallas as pl
from jax.experimental.pallas import tpu as pltpu
from jax.experimental.pallas import tpu_sc as plsc

B = 16384
K = 8192
D = 256
COMMIT = 0.25

BM = 1024
NB = B // BM

NC = 2
NS = 16
NW = NC * NS
ROWS_PER_W = B // NW
CHUNK = 256


def _dist_body(z_ref, cb_ref, idx_ref, loss_ref, acc_ref, esq_ref):
    i = pl.program_id(0)
    z = z_ref[...]

    @pl.when(i == 0)
    def _():
        cb = cb_ref[...]
        esq_ref[0, :] = jnp.sum(cb ** 2, axis=-1)
        acc_ref[0] = 0.0

    z_sq = jnp.sum(z ** 2, axis=-1, keepdims=True)
    e_sq = esq_ref[...]
    cols = lax.broadcasted_iota(jnp.int32, (1, K), 1).astype(jnp.float32)
    z2 = z + z
    NS_K = 4
    KH = K // NS_K
    dist = []
    for h in range(NS_K):
        ze2_h = lax.dot_general(z2, cb_ref[h * KH:(h + 1) * KH, :],
                                (((1,), (1,)), ((), ())),
                                preferred_element_type=jnp.float32)
        dist.append(z_sq + e_sq[:, h * KH:(h + 1) * KH] - ze2_h)
    mind_h = [jnp.min(dh, axis=-1) for dh in dist]
    mind = functools.reduce(jnp.minimum, mind_h)
    hit = [jnp.where(dist[h] == mind[:, None],
                     cols[:, h * KH:(h + 1) * KH], float(K))
           for h in range(NS_K)]
    idx_h = [jnp.min(hh, axis=-1) for hh in hit]
    idx_ref[0, 0, :] = functools.reduce(jnp.minimum, idx_h).astype(jnp.int32)

    acc_ref[0] += jnp.sum(mind)

    @pl.when(i == NB - 1)
    def _():
        loss_ref[0, 0] = acc_ref[0] * ((1.0 + COMMIT) / (B * D))


_dist_call = pl.pallas_call(
    _dist_body,
    grid=(NB,),
    in_specs=[
        pl.BlockSpec((BM, D), lambda i: (i, 0)),
        pl.BlockSpec((K, D), lambda i: (0, 0)),
    ],
    out_specs=[
        pl.BlockSpec((1, 1, BM), lambda i: (i, 0, 0)),
        pl.BlockSpec(memory_space=pltpu.SMEM, block_shape=(1, 1),
                     index_map=lambda i: (0, 0)),
    ],
    out_shape=[
        jax.ShapeDtypeStruct((NB, 1, BM), jnp.int32),
        jax.ShapeDtypeStruct((1, 1), jnp.float32),
    ],
    scratch_shapes=[pltpu.SMEM((1,), jnp.float32),
                    pltpu.VMEM((1, K), jnp.float32)],
)


def _gather_body(cb_hbm, idx_hbm, out_hbm, idx_v, rows_v, sem):
    wid = lax.axis_index("s") * NC + lax.axis_index("c")
    base = wid * ROWS_PER_W
    for c in range(ROWS_PER_W // CHUNK):
        off = base + c * CHUNK
        pltpu.sync_copy(idx_hbm.at[pl.ds(off, CHUNK)], idx_v)
        pltpu.async_copy(cb_hbm.at[idx_v], rows_v, sem).wait()
        pltpu.sync_copy(rows_v, out_hbm.at[pl.ds(off, CHUNK)])


@functools.cache
def _gather_call():
    return functools.partial(
        pl.kernel,
        out_type=jax.ShapeDtypeStruct((B, D), jnp.float32),
        mesh=plsc.VectorSubcoreMesh(core_axis_name="c", subcore_axis_name="s"),
        scratch_types=[
            pltpu.VMEM((CHUNK,), jnp.int32),
            pltpu.VMEM((CHUNK, D), jnp.float32),
            pltpu.SemaphoreType.DMA,
        ],
    )(_gather_body)


def kernel(z, codebook):
    idx_blocks, loss = _dist_call(z, codebook)
    indices = idx_blocks.reshape(B)
    z_q_out = _gather_call()(codebook, indices)
    return (z_q_out, loss.reshape(()), indices)


# speedup vs baseline: 1.3219x; 1.0038x over previous
import functools

import jax
import jax.numpy as jnp
from jax import lax
from jax.experimental import pallas as pl
from jax.experimental.pallas import tpu as pltpu
from jax.experimental.pallas import tpu_sc as plsc

B = 16384
K = 8192
D = 256
COMMIT = 0.25

BM = 1024
NB = B // BM

NC = 2
NS = 16
NW = NC * NS
ROWS_PER_W = B // NW
CHUNK = 256


def _dist_body(z_ref, cb_ref, idx_ref, loss_ref, acc_ref, esq_ref):
    i = pl.program_id(0)
    z = z_ref[...]

    @pl.when(i == 0)
    def _():
        cb = cb_ref[...]
        esq_ref[0, :] = jnp.sum(cb ** 2, axis=-1)
        acc_ref[0] = 0.0

    z_sq = jnp.sum(z ** 2, axis=-1, keepdims=True)
    e_sq = esq_ref[...]
    cols = lax.broadcasted_iota(jnp.int32, (1, K), 1).astype(jnp.float32)
    z2 = z + z
    NS_K = 8
    KH = K // NS_K
    dist = []
    for h in range(NS_K):
        ze2_h = lax.dot_general(z2, cb_ref[h * KH:(h + 1) * KH, :],
                                (((1,), (1,)), ((), ())),
                                preferred_element_type=jnp.float32)
        dist.append(z_sq + e_sq[:, h * KH:(h + 1) * KH] - ze2_h)
    mind_h = [jnp.min(dh, axis=-1) for dh in dist]
    mind = functools.reduce(jnp.minimum, mind_h)
    hit = [jnp.where(dist[h] == mind[:, None],
                     cols[:, h * KH:(h + 1) * KH], float(K))
           for h in range(NS_K)]
    idx_h = [jnp.min(hh, axis=-1) for hh in hit]
    idx_ref[0, 0, :] = functools.reduce(jnp.minimum, idx_h).astype(jnp.int32)

    acc_ref[0] += jnp.sum(mind)

    @pl.when(i == NB - 1)
    def _():
        loss_ref[0, 0] = acc_ref[0] * ((1.0 + COMMIT) / (B * D))


_dist_call = pl.pallas_call(
    _dist_body,
    grid=(NB,),
    in_specs=[
        pl.BlockSpec((BM, D), lambda i: (i, 0)),
        pl.BlockSpec((K, D), lambda i: (0, 0)),
    ],
    out_specs=[
        pl.BlockSpec((1, 1, BM), lambda i: (i, 0, 0)),
        pl.BlockSpec(memory_space=pltpu.SMEM, block_shape=(1, 1),
                     index_map=lambda i: (0, 0)),
    ],
    out_shape=[
        jax.ShapeDtypeStruct((NB, 1, BM), jnp.int32),
        jax.ShapeDtypeStruct((1, 1), jnp.float32),
    ],
    scratch_shapes=[pltpu.SMEM((1,), jnp.float32),
                    pltpu.VMEM((1, K), jnp.float32)],
)


def _gather_body(cb_hbm, idx_hbm, out_hbm, idx_v, rows_v, sem):
    wid = lax.axis_index("s") * NC + lax.axis_index("c")
    base = wid * ROWS_PER_W
    for c in range(ROWS_PER_W // CHUNK):
        off = base + c * CHUNK
        pltpu.sync_copy(idx_hbm.at[pl.ds(off, CHUNK)], idx_v)
        pltpu.async_copy(cb_hbm.at[idx_v], rows_v, sem).wait()
        pltpu.sync_copy(rows_v, out_hbm.at[pl.ds(off, CHUNK)])


@functools.cache
def _gather_call():
    return functools.partial(
        pl.kernel,
        out_type=jax.ShapeDtypeStruct((B, D), jnp.float32),
        mesh=plsc.VectorSubcoreMesh(core_axis_name="c", subcore_axis_name="s"),
        scratch_types=[
            pltpu.VMEM((CHUNK,), jnp.int32),
            pltpu.VMEM((CHUNK, D), jnp.float32),
            pltpu.SemaphoreType.DMA,
        ],
    )(_gather_body)


def kernel(z, codebook):
    idx_blocks, loss = _dist_call(z, codebook)
    indices = idx_blocks.reshape(B)
    z_q_out = _gather_call()(codebook, indices)
    return (z_q_out, loss.reshape(()), indices)
